# Initial kernel scaffold; baseline (speedup 1.0000x reference)
import functools

import jax
import jax.numpy as jnp
from jax import lax
from jax.experimental import pallas as pl
from jax.experimental.pallas import tpu as pltpu
from jax.experimental.pallas import tpu_sc as plsc

N = 10000
D_IN = 256
H1, C1 = 8, 16
F1 = H1 * C1
AUG1 = F1 + 16
AUG2 = 16
NP = 10240
NC, NS = 2, 16
NW = NC * NS
CH = 128
BR = 1024


def _prep1_body(x_ref, w1_ref, asm_ref, adm_ref, haug_ref, adst_ref):
    h = jnp.dot(x_ref[...], w1_ref[...], preferred_element_type=jnp.float32)
    asrc = jnp.dot(h, asm_ref[...], preferred_element_type=jnp.float32)
    adst = jnp.dot(h, adm_ref[...], preferred_element_type=jnp.float32)
    z8 = jnp.zeros_like(asrc)
    haug_ref[...] = jnp.concatenate([h, asrc, z8], axis=1)
    adst_ref[...] = jnp.concatenate([adst, z8], axis=1)


def _prep1(xp, W1, AS, AD):
    grid = NP // BR
    return pl.pallas_call(
        _prep1_body,
        grid=(grid,),
        in_specs=[
            pl.BlockSpec((BR, D_IN), lambda i: (i, 0)),
            pl.BlockSpec((D_IN, F1), lambda i: (0, 0)),
            pl.BlockSpec((F1, H1), lambda i: (0, 0)),
            pl.BlockSpec((F1, H1), lambda i: (0, 0)),
        ],
        out_specs=[
            pl.BlockSpec((BR, AUG1), lambda i: (i, 0)),
            pl.BlockSpec((BR, 16), lambda i: (i, 0)),
        ],
        out_shape=[
            jax.ShapeDtypeStruct((NP, AUG1), jnp.float32),
            jax.ShapeDtypeStruct((NP, 16), jnp.float32),
        ],
    )(xp, W1, AS, AD)


def _edge_sc1(haug, adstt, src, dst, ept):
    mesh = plsc.VectorSubcoreMesh(core_axis_name="c", subcore_axis_name="s")
    rpt = NP // NS

    @functools.partial(
        pl.kernel,
        out_type=jax.ShapeDtypeStruct((NC * NP, AUG1), jnp.float32),
        mesh=mesh,
        scratch_types=[
            pltpu.VMEM((CH,), jnp.int32),
            pltpu.VMEM((CH,), jnp.int32),
            pltpu.VMEM((CH, AUG1), jnp.float32),
            pltpu.VMEM((CH, 16), jnp.float32),
            pltpu.VMEM((CH, AUG1), jnp.float32),
            pltpu.VMEM((16,), jnp.float32),
            pltpu.VMEM_SHARED((NP, AUG1), jnp.float32),
            pltpu.SemaphoreType.DMA,
            pltpu.SemaphoreType.DMA,
        ],
    )
    def k(haug_hbm, adst_hbm, src_hbm, dst_hbm, out_hbm,
          srci_v, dsti_v, rows_v, rowd_v, msg_v, wbuf_v, acc_sh, sem, semd):
        c = lax.axis_index("c")
        s = lax.axis_index("s")
        wid = c * NS + s

        nblk = AUG1 // 16
        zero16 = jnp.zeros((16,), jnp.float32)

        def zrow(r, carry):
            for kk in range(nblk):
                msg_v[r, pl.ds(kk * 16, 16)] = zero16
            return carry
        lax.fori_loop(0, CH, zrow, 0)
        for j in range(rpt // CH):
            pltpu.sync_copy(msg_v, acc_sh.at[pl.ds(s * rpt + j * CH, CH)])
        plsc.subcore_barrier()

        iota = lax.iota(jnp.int32, 16)
        mask8 = jnp.where(iota < H1, 1.0, 0.0).astype(jnp.float32)
        bidx = [jnp.full((16,), kk, jnp.int32) for kk in range(H1)]
        base_w = wid * ept

        def chunk(g, carry):
            base = base_w + g * CH
            pltpu.sync_copy(src_hbm.at[pl.ds(base, CH)], srci_v)
            pltpu.sync_copy(dst_hbm.at[pl.ds(base, CH)], dsti_v)
            cp1 = pltpu.async_copy(haug_hbm.at[srci_v], rows_v, sem)
            cp2 = pltpu.async_copy(adst_hbm.at[dsti_v], rowd_v, semd)
            cp1.wait()
            cp2.wait()

            def edge(i, ecarry):
                asrc = rows_v[i, pl.ds(F1, 16)]
                adst = rowd_v[i, pl.ds(0, 16)]
                e = asrc + adst
                e = jnp.where(e >= 0.0, e, e * 0.2)
                w = jnp.exp(e)
                wbuf_v[...] = w
                for kk in range(H1):
                    wk = plsc.load_gather(wbuf_v, [bidx[kk]])
                    msg_v[i, pl.ds(kk * 16, 16)] = wk * rows_v[i, pl.ds(kk * 16, 16)]
                msg_v[i, pl.ds(F1, 16)] = w * mask8
                return ecarry
            lax.fori_loop(0, CH, edge, 0)
            pltpu.sync_copy(msg_v, acc_sh.at[dsti_v], add=True)
            return carry
        lax.fori_loop(0, ept // CH, chunk, 0)

        plsc.subcore_barrier()
        for j in range(rpt // CH):
            off = s * rpt + j * CH
            pltpu.sync_copy(acc_sh.at[pl.ds(off, CH)],
                            out_hbm.at[pl.ds(c * NP + off, CH)])

    return k(haug, adstt, src, dst)


def _edge_sc2(haug2, adstt2, src, dst, ept):
    mesh = plsc.VectorSubcoreMesh(core_axis_name="c", subcore_axis_name="s")
    rpt = NP // NS

    @functools.partial(
        pl.kernel,
        out_type=jax.ShapeDtypeStruct((NC * NP, AUG2), jnp.float32),
        mesh=mesh,
        scratch_types=[
            pltpu.VMEM((CH,), jnp.int32),
            pltpu.VMEM((CH,), jnp.int32),
            pltpu.VMEM((CH, AUG2), jnp.float32),
            pltpu.VMEM((CH, AUG2), jnp.float32),
            pltpu.VMEM((CH, AUG2), jnp.float32),
            pltpu.VMEM_SHARED((NP, AUG2), jnp.float32),
            pltpu.SemaphoreType.DMA,
            pltpu.SemaphoreType.DMA,
        ],
    )
    def k(haug_hbm, adst_hbm, src_hbm, dst_hbm, out_hbm,
          srci_v, dsti_v, rows_v, rowd_v, msg_v, acc_sh, sem, semd):
        c = lax.axis_index("c")
        s = lax.axis_index("s")
        wid = c * NS + s

        zero16 = jnp.zeros((16,), jnp.float32)

        def zrow(r, carry):
            msg_v[r, pl.ds(0, 16)] = zero16
            return carry
        lax.fori_loop(0, CH, zrow, 0)
        for j in range(rpt // CH):
            pltpu.sync_copy(msg_v, acc_sh.at[pl.ds(s * rpt + j * CH, CH)])
        plsc.subcore_barrier()

        iota = lax.iota(jnp.int32, 16)
        c0 = jnp.full((16,), 0, jnp.int32)
        c2 = jnp.full((16,), 2, jnp.int32)
        base_w = wid * ept

        def chunk(g, carry):
            base = base_w + g * CH
            pltpu.sync_copy(src_hbm.at[pl.ds(base, CH)], srci_v)
            pltpu.sync_copy(dst_hbm.at[pl.ds(base, CH)], dsti_v)
            cp1 = pltpu.async_copy(haug_hbm.at[srci_v], rows_v, sem)
            cp2 = pltpu.async_copy(adst_hbm.at[dsti_v], rowd_v, semd)
            cp1.wait()
            cp2.wait()

            def edge(i, ecarry):
                iv = jnp.full((16,), i, jnp.int32)
                asb = plsc.load_gather(rows_v, [iv, c2])
                adb = plsc.load_gather(rowd_v, [iv, c0])
                e = asb + adb
                e = jnp.where(e >= 0.0, e, e * 0.2)
                w = jnp.exp(e)
                rs = rows_v[i, pl.ds(0, 16)]
                sel = jnp.where(iota == 2, 1.0, rs)
                msg_v[i, pl.ds(0, 16)] = w * sel
                return ecarry
            lax.fori_loop(0, CH, edge, 0)
            pltpu.sync_copy(msg_v, acc_sh.at[dsti_v], add=True)
            return carry
        lax.fori_loop(0, ept // CH, chunk, 0)

        plsc.subcore_barrier()
        for j in range(rpt // CH):
            off = s * rpt + j * CH
            pltpu.sync_copy(acc_sh.at[pl.ds(off, CH)],
                            out_hbm.at[pl.ds(c * NP + off, CH)])

    return k(haug2, adstt2, src, dst)


def _mid_body(p0_ref, p1_ref, exp8_ref, b1_ref, w2_ref, a2s_ref, a2d_ref,
              haug2_ref, adst2_ref):
    ssum = p0_ref[...] + p1_ref[...]
    num = ssum[:, :F1]
    den = ssum[:, F1:F1 + H1]
    rec = 1.0 / (den + 1e-16)
    rec128 = jnp.dot(rec, exp8_ref[...], preferred_element_type=jnp.float32)
    o1 = num * rec128 + b1_ref[...]
    g = jnp.where(o1 > 0.0, o1, jnp.expm1(o1))
    h2 = jnp.dot(g, w2_ref[...], preferred_element_type=jnp.float32)
    s2 = jnp.dot(h2, a2s_ref[...], preferred_element_type=jnp.float32)
    d2 = jnp.dot(h2, a2d_ref[...], preferred_element_type=jnp.float32)
    zb = jnp.zeros((h2.shape[0], 13), jnp.float32)
    zc = jnp.zeros((h2.shape[0], 15), jnp.float32)
    haug2_ref[...] = jnp.concatenate([h2, s2, zb], axis=1)
    adst2_ref[...] = jnp.concatenate([d2, zc], axis=1)


def _mid(part1, EXP8, b1r, W2, a2s, a2d):
    grid = NP // BR
    return pl.pallas_call(
        _mid_body,
        grid=(grid,),
        in_specs=[
            pl.BlockSpec((BR, AUG1), lambda i: (i, 0)),
            pl.BlockSpec((BR, AUG1), lambda i: (i + NP // BR, 0)),
            pl.BlockSpec((H1, F1), lambda i: (0, 0)),
            pl.BlockSpec((1, F1), lambda i: (0, 0)),
            pl.BlockSpec((F1, 2), lambda i: (0, 0)),
            pl.BlockSpec((2, 1), lambda i: (0, 0)),
            pl.BlockSpec((2, 1), lambda i: (0, 0)),
        ],
        out_specs=[
            pl.BlockSpec((BR, AUG2), lambda i: (i, 0)),
            pl.BlockSpec((BR, AUG2), lambda i: (i, 0)),
        ],
        out_shape=[
            jax.ShapeDtypeStruct((NP, AUG2), jnp.float32),
            jax.ShapeDtypeStruct((NP, AUG2), jnp.float32),
        ],
    )(part1, part1, EXP8, b1r, W2, a2s, a2d)


def _fin_body(p0_ref, p1_ref, b2_ref, out_ref):
    ssum = p0_ref[...] + p1_ref[...]
    out_ref[...] = ssum[:, 0:2] / (ssum[:, 2:3] + 1e-16) + b2_ref[...]


def _fin(part2, b2r):
    grid = NP // BR
    return pl.pallas_call(
        _fin_body,
        grid=(grid,),
        in_specs=[
            pl.BlockSpec((BR, AUG2), lambda i: (i, 0)),
            pl.BlockSpec((BR, AUG2), lambda i: (i + NP // BR, 0)),
            pl.BlockSpec((1, 2), lambda i: (0, 0)),
        ],
        out_specs=pl.BlockSpec((BR, 2), lambda i: (i, 0)),
        out_shape=jax.ShapeDtypeStruct((NP, 2), jnp.float32),
    )(part2, part2, b2r)


def kernel(x, edge_index, W1, a1_src, a1_dst, b1, W2, a2_src, a2_dst, b2):
    E = edge_index.shape[1]
    loops = jnp.arange(N, dtype=jnp.int32)
    etot = E + N
    ept = -(-etot // (NW * CH)) * CH
    pad = ept * NW - etot
    junk = jnp.full((pad,), N, jnp.int32)
    src = jnp.concatenate([edge_index[0].astype(jnp.int32), loops, junk])
    dst = jnp.concatenate([edge_index[1].astype(jnp.int32), loops, junk])

    xp = jnp.pad(x, ((0, NP - N), (0, 0)))
    eye = jnp.eye(H1, dtype=jnp.float32)
    AS = (a1_src[0][:, :, None] * eye[:, None, :]).reshape(F1, H1)
    AD = (a1_dst[0][:, :, None] * eye[:, None, :]).reshape(F1, H1)

    haug1, adst1 = _prep1(xp, W1, AS, AD)
    part1 = _edge_sc1(haug1, adst1, src, dst, ept)

    EXP8 = (jnp.arange(F1)[None, :] // C1 == jnp.arange(H1)[:, None]
            ).astype(jnp.float32)
    haug2, adst2 = _mid(part1, EXP8, b1.reshape(1, F1), W2,
                        a2_src.reshape(2, 1), a2_dst.reshape(2, 1))
    part2 = _edge_sc2(haug2, adst2, src, dst, ept)
    outp = _fin(part2, b2.reshape(1, 2))
    return outp[:N]


# speedup vs baseline: 46.6120x; 46.6120x over previous
import functools

import jax
import jax.numpy as jnp
from jax import lax
from jax.experimental import pallas as pl
from jax.experimental.pallas import tpu as pltpu
from jax.experimental.pallas import tpu_sc as plsc

N = 10000
D_IN = 256
H1, C1 = 8, 16
F1 = H1 * C1
AUG1 = F1 + 16
AUG2 = 16
NP = 10208
NC, NS = 2, 16
NW = NC * NS
CH = 128
BR = 928


def _prep1_body(x_ref, w1_ref, asm_ref, adm_ref, haug_ref, adst_ref):
    h = jnp.dot(x_ref[...], w1_ref[...], preferred_element_type=jnp.float32)
    asrc = jnp.dot(h, asm_ref[...], preferred_element_type=jnp.float32)
    adst = jnp.dot(h, adm_ref[...], preferred_element_type=jnp.float32)
    z8 = jnp.zeros_like(asrc)
    haug_ref[...] = jnp.concatenate([h, asrc, z8], axis=1)
    adst_ref[...] = jnp.concatenate([adst, z8], axis=1)


def _prep1(xp, W1, AS, AD):
    grid = NP // BR
    return pl.pallas_call(
        _prep1_body,
        grid=(grid,),
        in_specs=[
            pl.BlockSpec((BR, D_IN), lambda i: (i, 0)),
            pl.BlockSpec((D_IN, F1), lambda i: (0, 0)),
            pl.BlockSpec((F1, H1), lambda i: (0, 0)),
            pl.BlockSpec((F1, H1), lambda i: (0, 0)),
        ],
        out_specs=[
            pl.BlockSpec((BR, AUG1), lambda i: (i, 0)),
            pl.BlockSpec((BR, 16), lambda i: (i, 0)),
        ],
        out_shape=[
            jax.ShapeDtypeStruct((NP, AUG1), jnp.float32),
            jax.ShapeDtypeStruct((NP, 16), jnp.float32),
        ],
    )(xp, W1, AS, AD)


def _edge_sc1(haug, adstt, src, dst, ept):
    mesh = plsc.VectorSubcoreMesh(core_axis_name="c", subcore_axis_name="s",
                                  num_cores=NC, num_subcores=NS)
    rpt = NP // NS

    @functools.partial(
        pl.kernel,
        out_type=jax.ShapeDtypeStruct((NC * NP, AUG1), jnp.float32),
        mesh=mesh,
        scratch_types=[
            pltpu.VMEM((CH,), jnp.int32),
            pltpu.VMEM((CH,), jnp.int32),
            pltpu.VMEM((CH, AUG1), jnp.float32),
            pltpu.VMEM((CH, 16), jnp.float32),
            pltpu.VMEM((CH, AUG1), jnp.float32),
            pltpu.VMEM_SHARED((NP, AUG1), jnp.float32),
            pltpu.SemaphoreType.DMA,
            pltpu.SemaphoreType.DMA,
        ],
        compiler_params=pltpu.CompilerParams(use_tc_tiling_on_sc=False),
    )
    def k(haug_hbm, adst_hbm, src_hbm, dst_hbm, out_hbm,
          srci_v, dsti_v, rows_v, rowd_v, msg_v, acc_sh, sem, semd):
        c = lax.axis_index("c")
        s = lax.axis_index("s")
        wid = c * NS + s

        nblk = AUG1 // 16
        zero16 = jnp.zeros((16,), jnp.float32)

        def zrow(r, carry):
            for kk in range(nblk):
                msg_v[r, pl.ds(kk * 16, 16)] = zero16
            return carry
        lax.fori_loop(0, CH, zrow, 0)
        off0 = 0
        for sz in ([CH] * (rpt // CH) + ([rpt % CH] if rpt % CH else [])):
            pltpu.sync_copy(msg_v.at[pl.ds(0, sz)],
                            acc_sh.at[pl.ds(s * rpt + off0, sz)])
            off0 += sz
        plsc.subcore_barrier()

        iota = lax.iota(jnp.int32, 16)
        mask8 = jnp.where(iota < H1, 1.0, 0.0).astype(jnp.float32)
        base_w = wid * ept

        def chunk(g, carry):
            base = base_w + g * CH
            pltpu.sync_copy(src_hbm.at[pl.ds(base, CH)], srci_v)
            pltpu.sync_copy(dst_hbm.at[pl.ds(base, CH)], dsti_v)
            cp1 = pltpu.async_copy(haug_hbm.at[srci_v], rows_v, sem)
            cp2 = pltpu.async_copy(adst_hbm.at[dsti_v], rowd_v, semd)
            cp1.wait()
            cp2.wait()

            def edge(i, ecarry):
                asrc = rows_v[i, pl.ds(F1, 16)]
                adst = rowd_v[i, pl.ds(0, 16)]
                e = asrc + adst
                e = jnp.where(e >= 0.0, e, e * 0.2)
                w = jnp.exp(e)
                for kk in range(H1):
                    msg_v[i, pl.ds(kk * 16, 16)] = w[kk] * rows_v[i, pl.ds(kk * 16, 16)]
                msg_v[i, pl.ds(F1, 16)] = w * mask8
                return ecarry
            lax.fori_loop(0, CH, edge, 0)
            pltpu.sync_copy(msg_v, acc_sh.at[dsti_v], add=True)
            return carry
        lax.fori_loop(0, ept // CH, chunk, 0)

        plsc.subcore_barrier()
        off1 = 0
        for sz in ([CH] * (rpt // CH) + ([rpt % CH] if rpt % CH else [])):
            off = s * rpt + off1
            pltpu.sync_copy(acc_sh.at[pl.ds(off, sz)],
                            out_hbm.at[pl.ds(c * NP + off, sz)])
            off1 += sz

    return k(haug, adstt, src, dst)


def _edge_sc2(haug2, adstt2, src, dst, ept):
    mesh = plsc.VectorSubcoreMesh(core_axis_name="c", subcore_axis_name="s",
                                  num_cores=NC, num_subcores=NS)
    rpt = NP // NS

    @functools.partial(
        pl.kernel,
        out_type=jax.ShapeDtypeStruct((NC * NP, AUG2), jnp.float32),
        mesh=mesh,
        scratch_types=[
            pltpu.VMEM((CH,), jnp.int32),
            pltpu.VMEM((CH,), jnp.int32),
            pltpu.VMEM((CH, AUG2), jnp.float32),
            pltpu.VMEM((CH, AUG2), jnp.float32),
            pltpu.VMEM((CH, AUG2), jnp.float32),
            pltpu.VMEM_SHARED((NP, AUG2), jnp.float32),
            pltpu.SemaphoreType.DMA,
            pltpu.SemaphoreType.DMA,
        ],
        compiler_params=pltpu.CompilerParams(use_tc_tiling_on_sc=False),
    )
    def k(haug_hbm, adst_hbm, src_hbm, dst_hbm, out_hbm,
          srci_v, dsti_v, rows_v, rowd_v, msg_v, acc_sh, sem, semd):
        c = lax.axis_index("c")
        s = lax.axis_index("s")
        wid = c * NS + s

        zero16 = jnp.zeros((16,), jnp.float32)

        def zrow(r, carry):
            msg_v[r, pl.ds(0, 16)] = zero16
            return carry
        lax.fori_loop(0, CH, zrow, 0)
        off0 = 0
        for sz in ([CH] * (rpt // CH) + ([rpt % CH] if rpt % CH else [])):
            pltpu.sync_copy(msg_v.at[pl.ds(0, sz)],
                            acc_sh.at[pl.ds(s * rpt + off0, sz)])
            off0 += sz
        plsc.subcore_barrier()

        iota = lax.iota(jnp.int32, 16)
        base_w = wid * ept

        def chunk(g, carry):
            base = base_w + g * CH
            pltpu.sync_copy(src_hbm.at[pl.ds(base, CH)], srci_v)
            pltpu.sync_copy(dst_hbm.at[pl.ds(base, CH)], dsti_v)
            cp1 = pltpu.async_copy(haug_hbm.at[srci_v], rows_v, sem)
            cp2 = pltpu.async_copy(adst_hbm.at[dsti_v], rowd_v, semd)
            cp1.wait()
            cp2.wait()

            def edge(i, ecarry):
                rs = rows_v[i, pl.ds(0, 16)]
                rd = rowd_v[i, pl.ds(0, 16)]
                ev = (rs[2] + rd[0]) + jnp.zeros((16,), jnp.float32)
                ev = jnp.where(ev >= 0.0, ev, ev * 0.2)
                w = jnp.exp(ev)
                sel = jnp.where(iota == 2, 1.0, rs)
                msg_v[i, pl.ds(0, 16)] = w * sel
                return ecarry
            lax.fori_loop(0, CH, edge, 0)
            pltpu.sync_copy(msg_v, acc_sh.at[dsti_v], add=True)
            return carry
        lax.fori_loop(0, ept // CH, chunk, 0)

        plsc.subcore_barrier()
        off1 = 0
        for sz in ([CH] * (rpt // CH) + ([rpt % CH] if rpt % CH else [])):
            off = s * rpt + off1
            pltpu.sync_copy(acc_sh.at[pl.ds(off, sz)],
                            out_hbm.at[pl.ds(c * NP + off, sz)])
            off1 += sz

    return k(haug2, adstt2, src, dst)


def _mid_body(p0_ref, p1_ref, exp8_ref, b1_ref, w2_ref, a2s_ref, a2d_ref,
              haug2_ref, adst2_ref):
    ssum = p0_ref[...] + p1_ref[...]
    num = ssum[:, :F1]
    den = ssum[:, F1:F1 + H1]
    rec = 1.0 / (den + 1e-16)
    rec128 = jnp.dot(rec, exp8_ref[...], preferred_element_type=jnp.float32)
    o1 = num * rec128 + b1_ref[...]
    g = jnp.where(o1 > 0.0, o1, jnp.exp(o1) - 1.0)
    h2 = jnp.dot(g, w2_ref[...], preferred_element_type=jnp.float32)
    s2 = jnp.dot(h2, a2s_ref[...], preferred_element_type=jnp.float32)
    d2 = jnp.dot(h2, a2d_ref[...], preferred_element_type=jnp.float32)
    zb = jnp.zeros((h2.shape[0], 13), jnp.float32)
    zc = jnp.zeros((h2.shape[0], 15), jnp.float32)
    haug2_ref[...] = jnp.concatenate([h2, s2, zb], axis=1)
    adst2_ref[...] = jnp.concatenate([d2, zc], axis=1)


def _mid(part1, EXP8, b1r, W2, a2s, a2d):
    grid = NP // BR
    return pl.pallas_call(
        _mid_body,
        grid=(grid,),
        in_specs=[
            pl.BlockSpec((BR, AUG1), lambda i: (i, 0)),
            pl.BlockSpec((BR, AUG1), lambda i: (i + NP // BR, 0)),
            pl.BlockSpec((H1, F1), lambda i: (0, 0)),
            pl.BlockSpec((1, F1), lambda i: (0, 0)),
            pl.BlockSpec((F1, 2), lambda i: (0, 0)),
            pl.BlockSpec((2, 1), lambda i: (0, 0)),
            pl.BlockSpec((2, 1), lambda i: (0, 0)),
        ],
        out_specs=[
            pl.BlockSpec((BR, AUG2), lambda i: (i, 0)),
            pl.BlockSpec((BR, AUG2), lambda i: (i, 0)),
        ],
        out_shape=[
            jax.ShapeDtypeStruct((NP, AUG2), jnp.float32),
            jax.ShapeDtypeStruct((NP, AUG2), jnp.float32),
        ],
    )(part1, part1, EXP8, b1r, W2, a2s, a2d)


def _fin_body(p0_ref, p1_ref, b2_ref, out_ref):
    ssum = p0_ref[...] + p1_ref[...]
    out_ref[...] = ssum[:, 0:2] / (ssum[:, 2:3] + 1e-16) + b2_ref[...]


def _fin(part2, b2r):
    grid = NP // BR
    return pl.pallas_call(
        _fin_body,
        grid=(grid,),
        in_specs=[
            pl.BlockSpec((BR, AUG2), lambda i: (i, 0)),
            pl.BlockSpec((BR, AUG2), lambda i: (i + NP // BR, 0)),
            pl.BlockSpec((1, 2), lambda i: (0, 0)),
        ],
        out_specs=pl.BlockSpec((BR, 2), lambda i: (i, 0)),
        out_shape=jax.ShapeDtypeStruct((NP, 2), jnp.float32),
    )(part2, part2, b2r)


def kernel(x, edge_index, W1, a1_src, a1_dst, b1, W2, a2_src, a2_dst, b2):
    E = edge_index.shape[1]
    loops = jnp.arange(N, dtype=jnp.int32)
    etot = E + N
    ept = -(-etot // (NW * CH)) * CH
    pad = ept * NW - etot
    junk = jnp.full((pad,), N, jnp.int32)
    src = jnp.concatenate([edge_index[0].astype(jnp.int32), loops, junk])
    dst = jnp.concatenate([edge_index[1].astype(jnp.int32), loops, junk])

    xp = jnp.pad(x, ((0, NP - N), (0, 0)))
    eye = jnp.eye(H1, dtype=jnp.float32)
    AS = (a1_src[0][:, :, None] * eye[:, None, :]).reshape(F1, H1)
    AD = (a1_dst[0][:, :, None] * eye[:, None, :]).reshape(F1, H1)

    haug1, adst1 = _prep1(xp, W1, AS, AD)
    part1 = _edge_sc1(haug1, adst1, src, dst, ept)

    EXP8 = (jnp.arange(F1)[None, :] // C1 == jnp.arange(H1)[:, None]
            ).astype(jnp.float32)
    haug2, adst2 = _mid(part1, EXP8, b1.reshape(1, F1), W2,
                        a2_src.reshape(2, 1), a2_dst.reshape(2, 1))
    part2 = _edge_sc2(haug2, adst2, src, dst, ept)
    outp = _fin(part2, b2.reshape(1, 2))
    return outp[:N]


# speedup vs baseline: 77.0675x; 1.6534x over previous
import functools

import jax
import jax.numpy as jnp
from jax import lax
from jax.experimental import pallas as pl
from jax.experimental.pallas import tpu as pltpu
from jax.experimental.pallas import tpu_sc as plsc

N = 10000
D_IN = 256
H1, C1 = 8, 16
F1 = H1 * C1
AUG1 = F1 + 16
AUG2 = 16
NP = 10016
NC, NS = 2, 16
NW = NC * NS
CH = 64
BR = 2504


def _prep1_body(x_ref, w1_ref, asm_ref, adm_ref, haug_ref, adst_ref):
    h = jnp.dot(x_ref[...], w1_ref[...], preferred_element_type=jnp.float32)
    asrc = jnp.dot(h, asm_ref[...], preferred_element_type=jnp.float32)
    adst = jnp.dot(h, adm_ref[...], preferred_element_type=jnp.float32)
    z8 = jnp.zeros_like(asrc)
    haug_ref[...] = jnp.concatenate([h, asrc, z8], axis=1)
    adst_ref[...] = jnp.concatenate([adst, z8], axis=1)


def _prep1(xp, W1, AS, AD):
    grid = NP // BR
    return pl.pallas_call(
        _prep1_body,
        grid=(grid,),
        in_specs=[
            pl.BlockSpec((BR, D_IN), lambda i: (i, 0)),
            pl.BlockSpec((D_IN, F1), lambda i: (0, 0)),
            pl.BlockSpec((F1, H1), lambda i: (0, 0)),
            pl.BlockSpec((F1, H1), lambda i: (0, 0)),
        ],
        out_specs=[
            pl.BlockSpec((BR, AUG1), lambda i: (i, 0)),
            pl.BlockSpec((BR, 16), lambda i: (i, 0)),
        ],
        out_shape=[
            jax.ShapeDtypeStruct((NP, AUG1), jnp.float32),
            jax.ShapeDtypeStruct((NP, 16), jnp.float32),
        ],
    )(xp, W1, AS, AD)


def _edge_sc1(haug, adstt, src2, dst2, ept):
    mesh = plsc.VectorSubcoreMesh(core_axis_name="c", subcore_axis_name="s",
                                  num_cores=NC, num_subcores=NS)
    rpt = NP // NS
    nch = ept // CH

    @functools.partial(
        pl.kernel,
        out_type=jax.ShapeDtypeStruct((NC * NP, AUG1), jnp.float32),
        mesh=mesh,
        scratch_types=[
            pltpu.VMEM((nch, CH), jnp.int32),
            pltpu.VMEM((nch, CH), jnp.int32),
            pltpu.VMEM((2, CH, AUG1), jnp.float32),
            pltpu.VMEM((2, CH, 16), jnp.float32),
            pltpu.VMEM((CH, AUG1), jnp.float32),
            pltpu.VMEM_SHARED((NP, AUG1), jnp.float32),
            pltpu.SemaphoreType.DMA,
            pltpu.SemaphoreType.DMA,
            pltpu.SemaphoreType.DMA,
            pltpu.SemaphoreType.DMA,
            pltpu.SemaphoreType.DMA,
        ],
        compiler_params=pltpu.CompilerParams(use_tc_tiling_on_sc=False),
    )
    def k(haug_hbm, adst_hbm, src_hbm, dst_hbm, out_hbm,
          sia, dia, rows, rowd, msg, acc_sh,
          semg0, semg1, semd0, semd1, sems):
        c = lax.axis_index("c")
        s = lax.axis_index("s")
        wid = c * NS + s
        semg = [semg0, semg1]
        semd = [semd0, semd1]

        pltpu.sync_copy(src_hbm.at[pl.ds(wid * nch, nch)], sia)
        pltpu.sync_copy(dst_hbm.at[pl.ds(wid * nch, nch)], dia)

        zero16 = jnp.zeros((16,), jnp.float32)

        def zrow(r, carry):
            for kk in range(AUG1 // 16):
                msg[r, pl.ds(kk * 16, 16)] = zero16
            return carry
        lax.fori_loop(0, CH, zrow, 0)
        off0 = 0
        for sz in ([CH] * (rpt // CH) + ([rpt % CH] if rpt % CH else [])):
            pltpu.sync_copy(msg.at[pl.ds(0, sz)],
                            acc_sh.at[pl.ds(s * rpt + off0, sz)])
            off0 += sz
        plsc.subcore_barrier()

        iota = lax.iota(jnp.int32, 16)
        mask8 = jnp.where(iota < H1, 1.0, 0.0).astype(jnp.float32)

        for b in range(2):
            pltpu.async_copy(haug_hbm.at[sia.at[b]], rows.at[b], semg[b])
            pltpu.async_copy(adst_hbm.at[dia.at[b]], rowd.at[b], semd[b])

        def handle(g, b):
            geff = g + b
            pltpu.make_async_copy(haug_hbm.at[pl.ds(0, CH)], rows.at[b],
                                  semg[b]).wait()
            pltpu.make_async_copy(adst_hbm.at[pl.ds(0, CH)], rowd.at[b],
                                  semd[b]).wait()

            @pl.when(geff >= 1)
            def _():
                pltpu.make_async_copy(haug_hbm.at[pl.ds(0, CH)], msg,
                                      sems).wait()

            @plsc.parallel_loop(0, CH, unroll=4)
            def edge(i):
                asrc = rows[b, i, pl.ds(F1, 16)]
                adst = rowd[b, i, pl.ds(0, 16)]
                e = asrc + adst
                e = jnp.where(e >= 0.0, e, e * 0.2)
                w = jnp.exp(e)
                for kk in range(H1):
                    msg[i, pl.ds(kk * 16, 16)] = (
                        w[kk] * rows[b, i, pl.ds(kk * 16, 16)])
                msg[i, pl.ds(F1, 16)] = w * mask8

            pltpu.async_copy(msg, acc_sh.at[dia.at[geff]], sems,
                             add=True)

            @pl.when(geff + 2 < nch)
            def _():
                pltpu.async_copy(haug_hbm.at[sia.at[geff + 2]], rows.at[b],
                                 semg[b])
                pltpu.async_copy(adst_hbm.at[dia.at[geff + 2]], rowd.at[b],
                                 semd[b])

        @pl.loop(0, nch, step=2)
        def _(g):
            for b in range(2):
                handle(g, b)

        pltpu.make_async_copy(haug_hbm.at[pl.ds(0, CH)], msg,
                              sems).wait()
        plsc.subcore_barrier()
        off1 = 0
        for sz in ([CH] * (rpt // CH) + ([rpt % CH] if rpt % CH else [])):
            off = s * rpt + off1
            pltpu.sync_copy(acc_sh.at[pl.ds(off, sz)],
                            out_hbm.at[pl.ds(c * NP + off, sz)])
            off1 += sz

    return k(haug, adstt, src2, dst2)


def _edge_sc2(haug2, adstt2, src2, dst2, ept):
    mesh = plsc.VectorSubcoreMesh(core_axis_name="c", subcore_axis_name="s",
                                  num_cores=NC, num_subcores=NS)
    rpt = NP // NS
    nch = ept // CH

    @functools.partial(
        pl.kernel,
        out_type=jax.ShapeDtypeStruct((NC * NP, AUG2), jnp.float32),
        mesh=mesh,
        scratch_types=[
            pltpu.VMEM((nch, CH), jnp.int32),
            pltpu.VMEM((nch, CH), jnp.int32),
            pltpu.VMEM((2, CH, AUG2), jnp.float32),
            pltpu.VMEM((2, CH, AUG2), jnp.float32),
            pltpu.VMEM((CH, AUG2), jnp.float32),
            pltpu.VMEM_SHARED((NP, AUG2), jnp.float32),
            pltpu.SemaphoreType.DMA,
            pltpu.SemaphoreType.DMA,
            pltpu.SemaphoreType.DMA,
            pltpu.SemaphoreType.DMA,
            pltpu.SemaphoreType.DMA,
        ],
        compiler_params=pltpu.CompilerParams(use_tc_tiling_on_sc=False),
    )
    def k(haug_hbm, adst_hbm, src_hbm, dst_hbm, out_hbm,
          sia, dia, rows, rowd, msg, acc_sh,
          semg0, semg1, semd0, semd1, sems):
        c = lax.axis_index("c")
        s = lax.axis_index("s")
        wid = c * NS + s
        semg = [semg0, semg1]
        semd = [semd0, semd1]

        pltpu.sync_copy(src_hbm.at[pl.ds(wid * nch, nch)], sia)
        pltpu.sync_copy(dst_hbm.at[pl.ds(wid * nch, nch)], dia)

        zero16 = jnp.zeros((16,), jnp.float32)

        def zrow(r, carry):
            msg[r, pl.ds(0, 16)] = zero16
            return carry
        lax.fori_loop(0, CH, zrow, 0)
        off0 = 0
        for sz in ([CH] * (rpt // CH) + ([rpt % CH] if rpt % CH else [])):
            pltpu.sync_copy(msg.at[pl.ds(0, sz)],
                            acc_sh.at[pl.ds(s * rpt + off0, sz)])
            off0 += sz
        plsc.subcore_barrier()

        iota = lax.iota(jnp.int32, 16)

        for b in range(2):
            pltpu.async_copy(haug_hbm.at[sia.at[b]], rows.at[b], semg[b])
            pltpu.async_copy(adst_hbm.at[dia.at[b]], rowd.at[b], semd[b])

        def handle(g, b):
            geff = g + b
            pltpu.make_async_copy(haug_hbm.at[pl.ds(0, CH)], rows.at[b],
                                  semg[b]).wait()
            pltpu.make_async_copy(adst_hbm.at[pl.ds(0, CH)], rowd.at[b],
                                  semd[b]).wait()

            @pl.when(geff >= 1)
            def _():
                pltpu.make_async_copy(haug_hbm.at[pl.ds(0, CH)], msg,
                                      sems).wait()

            @plsc.parallel_loop(0, CH, unroll=4)
            def edge(i):
                rs = rows[b, i, pl.ds(0, 16)]
                rd = rowd[b, i, pl.ds(0, 16)]
                ev = (rs[2] + rd[0]) + jnp.zeros((16,), jnp.float32)
                ev = jnp.where(ev >= 0.0, ev, ev * 0.2)
                w = jnp.exp(ev)
                sel = jnp.where(iota == 2, 1.0, rs)
                msg[i, pl.ds(0, 16)] = w * sel

            pltpu.async_copy(msg, acc_sh.at[dia.at[geff]], sems,
                             add=True)

            @pl.when(geff + 2 < nch)
            def _():
                pltpu.async_copy(haug_hbm.at[sia.at[geff + 2]], rows.at[b],
                                 semg[b])
                pltpu.async_copy(adst_hbm.at[dia.at[geff + 2]], rowd.at[b],
                                 semd[b])

        @pl.loop(0, nch, step=2)
        def _(g):
            for b in range(2):
                handle(g, b)

        pltpu.make_async_copy(haug_hbm.at[pl.ds(0, CH)], msg,
                              sems).wait()
        plsc.subcore_barrier()
        off1 = 0
        for sz in ([CH] * (rpt // CH) + ([rpt % CH] if rpt % CH else [])):
            off = s * rpt + off1
            pltpu.sync_copy(acc_sh.at[pl.ds(off, sz)],
                            out_hbm.at[pl.ds(c * NP + off, sz)])
            off1 += sz

    return k(haug2, adstt2, src2, dst2)


def _mid_body(p0_ref, p1_ref, exp8_ref, b1_ref, w2_ref, a2s_ref, a2d_ref,
              haug2_ref, adst2_ref):
    ssum = p0_ref[...] + p1_ref[...]
    num = ssum[:, :F1]
    den = ssum[:, F1:F1 + H1]
    rec = 1.0 / (den + 1e-16)
    rec128 = jnp.dot(rec, exp8_ref[...], preferred_element_type=jnp.float32)
    o1 = num * rec128 + b1_ref[...]
    g = jnp.where(o1 > 0.0, o1, jnp.exp(o1) - 1.0)
    h2 = jnp.dot(g, w2_ref[...], preferred_element_type=jnp.float32)
    s2 = jnp.dot(h2, a2s_ref[...], preferred_element_type=jnp.float32)
    d2 = jnp.dot(h2, a2d_ref[...], preferred_element_type=jnp.float32)
    zb = jnp.zeros((h2.shape[0], 13), jnp.float32)
    zc = jnp.zeros((h2.shape[0], 15), jnp.float32)
    haug2_ref[...] = jnp.concatenate([h2, s2, zb], axis=1)
    adst2_ref[...] = jnp.concatenate([d2, zc], axis=1)


def _mid(part1, EXP8, b1r, W2, a2s, a2d):
    grid = NP // BR
    return pl.pallas_call(
        _mid_body,
        grid=(grid,),
        in_specs=[
            pl.BlockSpec((BR, AUG1), lambda i: (i, 0)),
            pl.BlockSpec((BR, AUG1), lambda i: (i + NP // BR, 0)),
            pl.BlockSpec((H1, F1), lambda i: (0, 0)),
            pl.BlockSpec((1, F1), lambda i: (0, 0)),
            pl.BlockSpec((F1, 2), lambda i: (0, 0)),
            pl.BlockSpec((2, 1), lambda i: (0, 0)),
            pl.BlockSpec((2, 1), lambda i: (0, 0)),
        ],
        out_specs=[
            pl.BlockSpec((BR, AUG2), lambda i: (i, 0)),
            pl.BlockSpec((BR, AUG2), lambda i: (i, 0)),
        ],
        out_shape=[
            jax.ShapeDtypeStruct((NP, AUG2), jnp.float32),
            jax.ShapeDtypeStruct((NP, AUG2), jnp.float32),
        ],
    )(part1, part1, EXP8, b1r, W2, a2s, a2d)


def _fin_body(p0_ref, p1_ref, b2_ref, out_ref):
    ssum = p0_ref[...] + p1_ref[...]
    out_ref[...] = ssum[:, 0:2] / (ssum[:, 2:3] + 1e-16) + b2_ref[...]


def _fin(part2, b2r):
    grid = NP // BR
    return pl.pallas_call(
        _fin_body,
        grid=(grid,),
        in_specs=[
            pl.BlockSpec((BR, AUG2), lambda i: (i, 0)),
            pl.BlockSpec((BR, AUG2), lambda i: (i + NP // BR, 0)),
            pl.BlockSpec((1, 2), lambda i: (0, 0)),
        ],
        out_specs=pl.BlockSpec((BR, 2), lambda i: (i, 0)),
        out_shape=jax.ShapeDtypeStruct((NP, 2), jnp.float32),
    )(part2, part2, b2r)


def kernel(x, edge_index, W1, a1_src, a1_dst, b1, W2, a2_src, a2_dst, b2):
    E = edge_index.shape[1]
    loops = jnp.arange(N, dtype=jnp.int32)
    etot = E + N
    ept = -(-etot // (NW * CH)) * CH
    pad = ept * NW - etot
    junk = jnp.full((pad,), N, jnp.int32)
    src = jnp.concatenate([edge_index[0].astype(jnp.int32), loops, junk]
                          ).reshape(-1, CH)
    dst = jnp.concatenate([edge_index[1].astype(jnp.int32), loops, junk]
                          ).reshape(-1, CH)

    xp = jnp.pad(x, ((0, NP - N), (0, 0)))
    eye = jnp.eye(H1, dtype=jnp.float32)
    AS = (a1_src[0][:, :, None] * eye[:, None, :]).reshape(F1, H1)
    AD = (a1_dst[0][:, :, None] * eye[:, None, :]).reshape(F1, H1)

    haug1, adst1 = _prep1(xp, W1, AS, AD)
    part1 = _edge_sc1(haug1, adst1, src, dst, ept)

    EXP8 = (jnp.arange(F1)[None, :] // C1 == jnp.arange(H1)[:, None]
            ).astype(jnp.float32)
    haug2, adst2 = _mid(part1, EXP8, b1.reshape(1, F1), W2,
                        a2_src.reshape(2, 1), a2_dst.reshape(2, 1))
    part2 = _edge_sc2(haug2, adst2, src, dst, ept)
    outp = _fin(part2, b2.reshape(1, 2))
    return outp[:N]
